# baseline (speedup 1.0000x reference)
import functools

import jax
import jax.numpy as jnp
from jax import lax
from jax.experimental import pallas as pl
from jax.experimental.pallas import tpu as pltpu
from jax.experimental.pallas import tpu_sc as plsc

VOCAB_ROWS = 1000000
EMBED = 64
NCORES = 2
NSUB = 16
NW = NCORES * NSUB

B_TOTAL = 4096 * 200
B_PER_W = B_TOTAL // NW
CHUNK = 800
PAIRS = B_PER_W // (2 * CHUNK)

TBLK = 2048
TGRID = (VOCAB_ROWS + TBLK - 1) // TBLK
TROWS = (TGRID - 1) * (TBLK // 2) + (VOCAB_ROWS - (TGRID - 1) * TBLK)
TVOCAB = 2 * TROWS


def _transpose_block(t64_ref, out_ref):
    x = t64_ref[...]
    eye = (lax.broadcasted_iota(jnp.int32, (EMBED, EMBED), 0)
           == lax.broadcasted_iota(jnp.int32, (EMBED, EMBED), 1)
           ).astype(jnp.float32)
    y = lax.dot_general(x, eye, (((0,), (0,)), ((), ())),
                        precision=lax.Precision.HIGHEST)
    out_ref[...] = jnp.concatenate([y[:TBLK // 2], y[TBLK // 2:]], axis=1)


_table_transpose = pl.pallas_call(
    _transpose_block,
    grid=(TGRID,),
    in_specs=[pl.BlockSpec((EMBED, TBLK), lambda i: (0, i))],
    out_specs=pl.BlockSpec((TBLK // 2, 2 * EMBED), lambda i: (i, 0)),
    out_shape=jax.ShapeDtypeStruct((TROWS, 2 * EMBED), jnp.float32),
)


@functools.partial(
    pl.kernel,
    mesh=plsc.VectorSubcoreMesh(core_axis_name="c", subcore_axis_name="s"),
    out_type=jax.ShapeDtypeStruct((B_TOTAL, 2 * EMBED), jnp.float32),
    compiler_params=pltpu.CompilerParams(use_tc_tiling_on_sc=False),
    scratch_types=[
        pltpu.VMEM((B_PER_W,), jnp.int32),
        pltpu.VMEM((CHUNK, EMBED), jnp.float32),
        pltpu.VMEM((CHUNK, EMBED), jnp.float32),
        pltpu.SemaphoreType.DMA,
        pltpu.SemaphoreType.DMA,
        pltpu.SemaphoreType.DMA,
        pltpu.SemaphoreType.DMA,
    ],
)
def _emb_lookup(table_hbm, idx_hbm, out_hbm,
                idx_all, buf0, buf1, gs0, gs1, ws0, ws1):
    wid = lax.axis_index("s") * NCORES + lax.axis_index("c")
    base = wid * B_PER_W

    pltpu.sync_copy(idx_hbm.at[pl.ds(base, B_PER_W)], idx_all)

    def gather(chunk, buf, sem):
        return pltpu.make_async_copy(
            table_hbm.at[idx_all.at[pl.ds(chunk * CHUNK, CHUNK)]], buf, sem)

    def writeback(chunk, buf, sem):
        return pltpu.make_async_copy(
            buf, out_hbm.at[pl.ds(base + chunk * CHUNK, CHUNK),
                            pl.ds(0, EMBED)], sem)

    gather(0, buf0, gs0).start()

    def body(j, _):
        c0 = 2 * j

        @pl.when(j > 0)
        def _():
            writeback(c0 - 1, buf1, ws1).wait()

        gather(c0 + 1, buf1, gs1).start()

        gather(c0, buf0, gs0).wait()
        writeback(c0, buf0, ws0).start()

        @pl.when(j < PAIRS - 1)
        def _():
            writeback(c0, buf0, ws0).wait()
            gather(c0 + 2, buf0, gs0).start()

        gather(c0 + 1, buf1, gs1).wait()
        writeback(c0 + 1, buf1, ws1).start()
        return 0

    lax.fori_loop(0, PAIRS, body, 0)

    last = 2 * PAIRS - 1
    writeback(last - 1, buf0, ws0).wait()
    writeback(last, buf1, ws1).wait()


def kernel(indices, table):
    t_lin = _table_transpose(table.T)
    half = TBLK // 2
    flat = ((indices & ~(TBLK - 1))
            | ((indices & (half - 1)) << 1)
            | ((indices >> (half.bit_length() - 1)) & 1)).reshape(-1)
    out = _emb_lookup(t_lin.reshape(TVOCAB, EMBED), flat)
    return out[:, :EMBED].reshape(indices.shape + (EMBED,))


# speedup vs baseline: 1.4060x; 1.4060x over previous
import functools

import jax
import jax.numpy as jnp
from jax import lax
from jax.experimental import pallas as pl
from jax.experimental.pallas import tpu as pltpu
from jax.experimental.pallas import tpu_sc as plsc

VOCAB_ROWS = 1000000
EMBED = 64
NCORES = 2
NSUB = 16
NW = NCORES * NSUB

B_TOTAL = 4096 * 200
B_PER_W = B_TOTAL // NW
CHUNK = 800
PAIRS = B_PER_W // (2 * CHUNK)

TBLK = 4096
TGRID = (VOCAB_ROWS + TBLK - 1) // TBLK
TROWS = (TGRID - 1) * (TBLK // 2) + (VOCAB_ROWS - (TGRID - 1) * TBLK)
TVOCAB = 2 * TROWS


def _transpose_block(t64_ref, out_ref):
    y = t64_ref[...].T
    out_ref[...] = jnp.concatenate([y[:TBLK // 2], y[TBLK // 2:]], axis=1)


_table_transpose = pl.pallas_call(
    _transpose_block,
    grid=(TGRID,),
    in_specs=[pl.BlockSpec((EMBED, TBLK), lambda i: (0, i))],
    out_specs=pl.BlockSpec((TBLK // 2, 2 * EMBED), lambda i: (i, 0)),
    out_shape=jax.ShapeDtypeStruct((TROWS, 2 * EMBED), jnp.float32),
)


@functools.partial(
    pl.kernel,
    mesh=plsc.VectorSubcoreMesh(core_axis_name="c", subcore_axis_name="s"),
    out_type=jax.ShapeDtypeStruct((B_TOTAL, 2 * EMBED), jnp.float32),
    compiler_params=pltpu.CompilerParams(use_tc_tiling_on_sc=False),
    scratch_types=[
        pltpu.VMEM((B_PER_W,), jnp.int32),
        pltpu.VMEM((CHUNK, EMBED), jnp.float32),
        pltpu.VMEM((CHUNK, EMBED), jnp.float32),
        pltpu.SemaphoreType.DMA,
        pltpu.SemaphoreType.DMA,
        pltpu.SemaphoreType.DMA,
        pltpu.SemaphoreType.DMA,
    ],
)
def _emb_lookup(table_hbm, idx_hbm, out_hbm,
                idx_all, buf0, buf1, gs0, gs1, ws0, ws1):
    wid = lax.axis_index("s") * NCORES + lax.axis_index("c")
    base = wid * B_PER_W

    pltpu.sync_copy(idx_hbm.at[pl.ds(base, B_PER_W)], idx_all)

    def gather(chunk, buf, sem):
        return pltpu.make_async_copy(
            table_hbm.at[idx_all.at[pl.ds(chunk * CHUNK, CHUNK)]], buf, sem)

    def writeback(chunk, buf, sem):
        return pltpu.make_async_copy(
            buf, out_hbm.at[pl.ds(base + chunk * CHUNK, CHUNK),
                            pl.ds(0, EMBED)], sem)

    gather(0, buf0, gs0).start()

    def body(j, _):
        c0 = 2 * j

        @pl.when(j > 0)
        def _():
            writeback(c0 - 1, buf1, ws1).wait()

        gather(c0 + 1, buf1, gs1).start()

        gather(c0, buf0, gs0).wait()
        writeback(c0, buf0, ws0).start()

        @pl.when(j < PAIRS - 1)
        def _():
            writeback(c0, buf0, ws0).wait()
            gather(c0 + 2, buf0, gs0).start()

        gather(c0 + 1, buf1, gs1).wait()
        writeback(c0 + 1, buf1, ws1).start()
        return 0

    lax.fori_loop(0, PAIRS, body, 0)

    last = 2 * PAIRS - 1
    writeback(last - 1, buf0, ws0).wait()
    writeback(last, buf1, ws1).wait()


def kernel(indices, table):
    t_lin = _table_transpose(table.T)
    half = TBLK // 2
    flat = ((indices & ~(TBLK - 1))
            | ((indices & (half - 1)) << 1)
            | ((indices >> (half.bit_length() - 1)) & 1)).reshape(-1)
    out = _emb_lookup(t_lin.reshape(TVOCAB, EMBED), flat)
    return out[:, :EMBED].reshape(indices.shape + (EMBED,))


# speedup vs baseline: 1.5584x; 1.1084x over previous
import functools

import jax
import jax.numpy as jnp
from jax import lax
from jax.experimental import pallas as pl
from jax.experimental.pallas import tpu as pltpu
from jax.experimental.pallas import tpu_sc as plsc

VOCAB_ROWS = 1000000
EMBED = 64
NCORES = 2
NSUB = 16
NW = NCORES * NSUB

B_TOTAL = 4096 * 200
B_PER_W = B_TOTAL // NW
CHUNK = 800
PAIRS = B_PER_W // (2 * CHUNK)

TBLK = 8192
TGRID = (VOCAB_ROWS + TBLK - 1) // TBLK
TROWS = (TGRID - 1) * (TBLK // 2) + (VOCAB_ROWS - (TGRID - 1) * TBLK)
TVOCAB = 2 * TROWS


def _transpose_block(t64_ref, out_ref):
    y = t64_ref[...].T
    out_ref[...] = jnp.concatenate([y[:TBLK // 2], y[TBLK // 2:]], axis=1)


_table_transpose = pl.pallas_call(
    _transpose_block,
    grid=(TGRID,),
    in_specs=[pl.BlockSpec((EMBED, TBLK), lambda i: (0, i))],
    out_specs=pl.BlockSpec((TBLK // 2, 2 * EMBED), lambda i: (i, 0)),
    out_shape=jax.ShapeDtypeStruct((TROWS, 2 * EMBED), jnp.float32),
)


@functools.partial(
    pl.kernel,
    mesh=plsc.VectorSubcoreMesh(core_axis_name="c", subcore_axis_name="s"),
    out_type=jax.ShapeDtypeStruct((B_TOTAL, 2 * EMBED), jnp.float32),
    compiler_params=pltpu.CompilerParams(use_tc_tiling_on_sc=False),
    scratch_types=[
        pltpu.VMEM((B_PER_W,), jnp.int32),
        pltpu.VMEM((CHUNK, EMBED), jnp.float32),
        pltpu.VMEM((CHUNK, EMBED), jnp.float32),
        pltpu.SemaphoreType.DMA,
        pltpu.SemaphoreType.DMA,
        pltpu.SemaphoreType.DMA,
        pltpu.SemaphoreType.DMA,
    ],
)
def _emb_lookup(table_hbm, idx_hbm, out_hbm,
                idx_all, buf0, buf1, gs0, gs1, ws0, ws1):
    wid = lax.axis_index("s") * NCORES + lax.axis_index("c")
    base = wid * B_PER_W

    pltpu.sync_copy(idx_hbm.at[pl.ds(base, B_PER_W)], idx_all)

    def gather(chunk, buf, sem):
        return pltpu.make_async_copy(
            table_hbm.at[idx_all.at[pl.ds(chunk * CHUNK, CHUNK)]], buf, sem)

    def writeback(chunk, buf, sem):
        return pltpu.make_async_copy(
            buf, out_hbm.at[pl.ds(base + chunk * CHUNK, CHUNK),
                            pl.ds(0, EMBED)], sem)

    gather(0, buf0, gs0).start()

    def body(j, _):
        c0 = 2 * j

        @pl.when(j > 0)
        def _():
            writeback(c0 - 1, buf1, ws1).wait()

        gather(c0 + 1, buf1, gs1).start()

        gather(c0, buf0, gs0).wait()
        writeback(c0, buf0, ws0).start()

        @pl.when(j < PAIRS - 1)
        def _():
            writeback(c0, buf0, ws0).wait()
            gather(c0 + 2, buf0, gs0).start()

        gather(c0 + 1, buf1, gs1).wait()
        writeback(c0 + 1, buf1, ws1).start()
        return 0

    lax.fori_loop(0, PAIRS, body, 0)

    last = 2 * PAIRS - 1
    writeback(last - 1, buf0, ws0).wait()
    writeback(last, buf1, ws1).wait()


def kernel(indices, table):
    t_lin = _table_transpose(table.T)
    half = TBLK // 2
    flat = ((indices & ~(TBLK - 1))
            | ((indices & (half - 1)) << 1)
            | ((indices >> (half.bit_length() - 1)) & 1)).reshape(-1)
    out = _emb_lookup(t_lin.reshape(TVOCAB, EMBED), flat)
    return out[:, :EMBED].reshape(indices.shape + (EMBED,))


# speedup vs baseline: 1.6625x; 1.0668x over previous
import functools

import jax
import jax.numpy as jnp
from jax import lax
from jax.experimental import pallas as pl
from jax.experimental.pallas import tpu as pltpu
from jax.experimental.pallas import tpu_sc as plsc

VOCAB_ROWS = 1000000
EMBED = 64
NCORES = 2
NSUB = 16
NW = NCORES * NSUB

B_TOTAL = 4096 * 200
B_PER_W = B_TOTAL // NW
CHUNK = 800
PAIRS = B_PER_W // (2 * CHUNK)

TBLK = 16384
TGRID = (VOCAB_ROWS + TBLK - 1) // TBLK
TROWS = (TGRID - 1) * (TBLK // 2) + (VOCAB_ROWS - (TGRID - 1) * TBLK)
TVOCAB = 2 * TROWS


def _transpose_block(t64_ref, out_ref):
    y = t64_ref[...].T
    out_ref[...] = jnp.concatenate([y[:TBLK // 2], y[TBLK // 2:]], axis=1)


_table_transpose = pl.pallas_call(
    _transpose_block,
    grid=(TGRID,),
    in_specs=[pl.BlockSpec((EMBED, TBLK), lambda i: (0, i))],
    out_specs=pl.BlockSpec((TBLK // 2, 2 * EMBED), lambda i: (i, 0)),
    out_shape=jax.ShapeDtypeStruct((TROWS, 2 * EMBED), jnp.float32),
)


@functools.partial(
    pl.kernel,
    mesh=plsc.VectorSubcoreMesh(core_axis_name="c", subcore_axis_name="s"),
    out_type=jax.ShapeDtypeStruct((B_TOTAL, 2 * EMBED), jnp.float32),
    compiler_params=pltpu.CompilerParams(use_tc_tiling_on_sc=False),
    scratch_types=[
        pltpu.VMEM((B_PER_W,), jnp.int32),
        pltpu.VMEM((CHUNK, EMBED), jnp.float32),
        pltpu.VMEM((CHUNK, EMBED), jnp.float32),
        pltpu.SemaphoreType.DMA,
        pltpu.SemaphoreType.DMA,
        pltpu.SemaphoreType.DMA,
        pltpu.SemaphoreType.DMA,
    ],
)
def _emb_lookup(table_hbm, idx_hbm, out_hbm,
                idx_all, buf0, buf1, gs0, gs1, ws0, ws1):
    wid = lax.axis_index("s") * NCORES + lax.axis_index("c")
    base = wid * B_PER_W

    pltpu.sync_copy(idx_hbm.at[pl.ds(base, B_PER_W)], idx_all)

    def gather(chunk, buf, sem):
        return pltpu.make_async_copy(
            table_hbm.at[idx_all.at[pl.ds(chunk * CHUNK, CHUNK)]], buf, sem)

    def writeback(chunk, buf, sem):
        return pltpu.make_async_copy(
            buf, out_hbm.at[pl.ds(base + chunk * CHUNK, CHUNK),
                            pl.ds(0, EMBED)], sem)

    gather(0, buf0, gs0).start()

    def body(j, _):
        c0 = 2 * j

        @pl.when(j > 0)
        def _():
            writeback(c0 - 1, buf1, ws1).wait()

        gather(c0 + 1, buf1, gs1).start()

        gather(c0, buf0, gs0).wait()
        writeback(c0, buf0, ws0).start()

        @pl.when(j < PAIRS - 1)
        def _():
            writeback(c0, buf0, ws0).wait()
            gather(c0 + 2, buf0, gs0).start()

        gather(c0 + 1, buf1, gs1).wait()
        writeback(c0 + 1, buf1, ws1).start()
        return 0

    lax.fori_loop(0, PAIRS, body, 0)

    last = 2 * PAIRS - 1
    writeback(last - 1, buf0, ws0).wait()
    writeback(last, buf1, ws1).wait()


def kernel(indices, table):
    t_lin = _table_transpose(table.T)
    half = TBLK // 2
    flat = ((indices & ~(TBLK - 1))
            | ((indices & (half - 1)) << 1)
            | ((indices >> (half.bit_length() - 1)) & 1)).reshape(-1)
    out = _emb_lookup(t_lin.reshape(TVOCAB, EMBED), flat)
    return out[:, :EMBED].reshape(indices.shape + (EMBED,))


# speedup vs baseline: 1.6970x; 1.0208x over previous
import functools

import jax
import jax.numpy as jnp
from jax import lax
from jax.experimental import pallas as pl
from jax.experimental.pallas import tpu as pltpu
from jax.experimental.pallas import tpu_sc as plsc

VOCAB_ROWS = 1000000
EMBED = 64
NCORES = 2
NSUB = 16
NW = NCORES * NSUB

B_TOTAL = 4096 * 200
B_PER_W = B_TOTAL // NW
CHUNK = 800
PAIRS = B_PER_W // (2 * CHUNK)

TBLK = 32768
TGRID = (VOCAB_ROWS + TBLK - 1) // TBLK
TROWS = (TGRID - 1) * (TBLK // 2) + (VOCAB_ROWS - (TGRID - 1) * TBLK)
TVOCAB = 2 * TROWS


def _transpose_block(t64_ref, out_ref):
    y = t64_ref[...].T
    out_ref[...] = jnp.concatenate([y[:TBLK // 2], y[TBLK // 2:]], axis=1)


_table_transpose = pl.pallas_call(
    _transpose_block,
    grid=(TGRID,),
    in_specs=[pl.BlockSpec((EMBED, TBLK), lambda i: (0, i))],
    out_specs=pl.BlockSpec((TBLK // 2, 2 * EMBED), lambda i: (i, 0)),
    out_shape=jax.ShapeDtypeStruct((TROWS, 2 * EMBED), jnp.float32),
)


@functools.partial(
    pl.kernel,
    mesh=plsc.VectorSubcoreMesh(core_axis_name="c", subcore_axis_name="s"),
    out_type=jax.ShapeDtypeStruct((B_TOTAL, 2 * EMBED), jnp.float32),
    compiler_params=pltpu.CompilerParams(use_tc_tiling_on_sc=False),
    scratch_types=[
        pltpu.VMEM((B_PER_W,), jnp.int32),
        pltpu.VMEM((CHUNK, EMBED), jnp.float32),
        pltpu.VMEM((CHUNK, EMBED), jnp.float32),
        pltpu.SemaphoreType.DMA,
        pltpu.SemaphoreType.DMA,
        pltpu.SemaphoreType.DMA,
        pltpu.SemaphoreType.DMA,
    ],
)
def _emb_lookup(table_hbm, idx_hbm, out_hbm,
                idx_all, buf0, buf1, gs0, gs1, ws0, ws1):
    wid = lax.axis_index("s") * NCORES + lax.axis_index("c")
    base = wid * B_PER_W

    pltpu.sync_copy(idx_hbm.at[pl.ds(base, B_PER_W)], idx_all)

    def gather(chunk, buf, sem):
        return pltpu.make_async_copy(
            table_hbm.at[idx_all.at[pl.ds(chunk * CHUNK, CHUNK)]], buf, sem)

    def writeback(chunk, buf, sem):
        return pltpu.make_async_copy(
            buf, out_hbm.at[pl.ds(base + chunk * CHUNK, CHUNK),
                            pl.ds(0, EMBED)], sem)

    gather(0, buf0, gs0).start()

    def body(j, _):
        c0 = 2 * j

        @pl.when(j > 0)
        def _():
            writeback(c0 - 1, buf1, ws1).wait()

        gather(c0 + 1, buf1, gs1).start()

        gather(c0, buf0, gs0).wait()
        writeback(c0, buf0, ws0).start()

        @pl.when(j < PAIRS - 1)
        def _():
            writeback(c0, buf0, ws0).wait()
            gather(c0 + 2, buf0, gs0).start()

        gather(c0 + 1, buf1, gs1).wait()
        writeback(c0 + 1, buf1, ws1).start()
        return 0

    lax.fori_loop(0, PAIRS, body, 0)

    last = 2 * PAIRS - 1
    writeback(last - 1, buf0, ws0).wait()
    writeback(last, buf1, ws1).wait()


def kernel(indices, table):
    t_lin = _table_transpose(table.T)
    half = TBLK // 2
    flat = ((indices & ~(TBLK - 1))
            | ((indices & (half - 1)) << 1)
            | ((indices >> (half.bit_length() - 1)) & 1)).reshape(-1)
    out = _emb_lookup(t_lin.reshape(TVOCAB, EMBED), flat)
    return out[:, :EMBED].reshape(indices.shape + (EMBED,))
